# initial kernel scaffold (unmeasured)
import jax
import jax.numpy as jnp
from jax import lax
from jax.experimental import pallas as pl
from jax.experimental.pallas import tpu as pltpu

N_DEV = 4
N_LAYERS = 3
N_EXCH = 2 * N_LAYERS


def kernel(x, Win0, Wout0, Win1, Wout1, Win2, Wout2):
    b, d = x.shape

    def body(x_ref, win0, wout0, win1, wout1, win2, wout2,
             out_ref, acc_ref, comm_ref, send_sems, recv_sems):
        my_pos = lax.axis_index("i")
        partner0 = my_pos ^ 1
        partner1 = 3 - my_pos

        wins = [win0, win1, win2]
        wouts = [wout0, wout1, wout2]

        cur = x_ref[...].astype(jnp.bfloat16)
        for l in range(N_LAYERS):
            w_in = wins[l][...].astype(jnp.bfloat16)
            w_out = wouts[l][...].astype(jnp.bfloat16)
            h = jnp.maximum(
                jnp.dot(cur, w_in, preferred_element_type=jnp.float32), 0.0
            ).astype(jnp.bfloat16)
            acc_ref[...] = jnp.dot(h, w_out, preferred_element_type=jnp.float32)
            for s in range(2):
                e = 2 * l + s
                partner = partner0 if s == 0 else partner1
                rdma = pltpu.make_async_remote_copy(
                    src_ref=acc_ref,
                    dst_ref=comm_ref.at[e],
                    send_sem=send_sems.at[e],
                    recv_sem=recv_sems.at[e],
                    device_id=(partner,),
                    device_id_type=pl.DeviceIdType.MESH,
                )
                rdma.start()
                rdma.wait()
                acc_ref[...] = acc_ref[...] + comm_ref[e]
            cur = acc_ref[...].astype(jnp.bfloat16)

        out_ref[...] = acc_ref[...]

    return pl.pallas_call(
        body,
        out_shape=jax.ShapeDtypeStruct((b, d), jnp.float32),
        in_specs=[pl.BlockSpec(memory_space=pltpu.VMEM)] * 7,
        out_specs=pl.BlockSpec(memory_space=pltpu.VMEM),
        scratch_shapes=[
            pltpu.VMEM((b, d), jnp.float32),
            pltpu.VMEM((N_EXCH, b, d), jnp.float32),
            pltpu.SemaphoreType.DMA((N_EXCH,)),
            pltpu.SemaphoreType.DMA((N_EXCH,)),
        ],
        compiler_params=pltpu.CompilerParams(collective_id=0),
    )(x, Win0, Wout0, Win1, Wout1, Win2, Wout2)


# baseline (device time: 60426 ns/iter reference)
import jax
import jax.numpy as jnp
from jax import lax
from jax.experimental import pallas as pl
from jax.experimental.pallas import tpu as pltpu

N_DEV = 4
N_LAYERS = 3
N_EXCH = 2 * N_LAYERS


def kernel(x, Win0, Wout0, Win1, Wout1, Win2, Wout2):
    b, d = x.shape

    def body(x_ref, win0, wout0, win1, wout1, win2, wout2,
             out_ref, acc_ref, comm_ref, send_sems, recv_sems):
        my_pos = lax.axis_index("i")
        partner0 = my_pos ^ 1
        partner1 = 3 - my_pos

        wins = [win0, win1, win2]
        wouts = [wout0, wout1, wout2]

        cur = x_ref[...].astype(jnp.bfloat16)
        for l in range(N_LAYERS):
            w_in = wins[l][...].astype(jnp.bfloat16)
            w_out = wouts[l][...].astype(jnp.bfloat16)
            h = jnp.maximum(
                jnp.dot(cur, w_in, preferred_element_type=jnp.float32), 0.0
            ).astype(jnp.bfloat16)
            acc_ref[...] = jnp.dot(h, w_out, preferred_element_type=jnp.float32)
            for s in range(2):
                e = 2 * l + s
                partner = partner0 if s == 0 else partner1
                rdma = pltpu.make_async_remote_copy(
                    src_ref=acc_ref,
                    dst_ref=comm_ref.at[e],
                    send_sem=send_sems.at[e],
                    recv_sem=recv_sems.at[e],
                    device_id=(partner,),
                    device_id_type=pl.DeviceIdType.MESH,
                )
                rdma.start()
                rdma.wait()
                acc_ref[...] = acc_ref[...] + comm_ref[e]
            cur = acc_ref[...].astype(jnp.bfloat16)

        out_ref[...] = acc_ref[...]

    return pl.pallas_call(
        body,
        out_shape=jax.ShapeDtypeStruct((b, d), jnp.float32),
        in_specs=[pl.BlockSpec(memory_space=pltpu.VMEM)] * 7,
        out_specs=pl.BlockSpec(memory_space=pltpu.VMEM),
        scratch_shapes=[
            pltpu.VMEM((b, d), jnp.float32),
            pltpu.VMEM((N_EXCH, b, d), jnp.float32),
            pltpu.SemaphoreType.DMA((N_EXCH,)),
            pltpu.SemaphoreType.DMA((N_EXCH,)),
        ],
    )(x, Win0, Wout0, Win1, Wout1, Win2, Wout2)


# device time: 43392 ns/iter; 1.3926x vs baseline; 1.3926x over previous
import jax
import jax.numpy as jnp
from jax import lax
from jax.experimental import pallas as pl
from jax.experimental.pallas import tpu as pltpu

N_DEV = 4
N_LAYERS = 3
N_EXCH = 2 * N_LAYERS


def kernel(x, Win0, Wout0, Win1, Wout1, Win2, Wout2):
    b, d = x.shape

    def body(x_ref, win0, wout0, win1, wout1, win2, wout2,
             out_ref, send_ref, comm_ref, send_sems, recv_sems):
        my_pos = lax.axis_index("i")
        partner0 = my_pos ^ 1
        partner1 = 3 - my_pos

        wins = [win0, win1, win2]
        wouts = [wout0, wout1, wout2]

        cur = x_ref[...].astype(jnp.bfloat16)
        for l in range(N_LAYERS):
            w_in = wins[l][...].astype(jnp.bfloat16)
            w_out = wouts[l][...].astype(jnp.bfloat16)
            h = jnp.maximum(
                jnp.dot(cur, w_in, preferred_element_type=jnp.float32), 0.0
            ).astype(jnp.bfloat16)
            acc = jnp.dot(h, w_out, preferred_element_type=jnp.float32)
            for s in range(2):
                e = 2 * l + s
                partner = partner0 if s == 0 else partner1
                send_ref[e] = acc.astype(jnp.bfloat16)
                rdma = pltpu.make_async_remote_copy(
                    src_ref=send_ref.at[e],
                    dst_ref=comm_ref.at[e],
                    send_sem=send_sems.at[e],
                    recv_sem=recv_sems.at[e],
                    device_id=(partner,),
                    device_id_type=pl.DeviceIdType.MESH,
                )
                rdma.start()
                rdma.wait()
                acc = acc + comm_ref[e].astype(jnp.float32)
            cur = acc.astype(jnp.bfloat16)

        out_ref[...] = acc

    return pl.pallas_call(
        body,
        out_shape=jax.ShapeDtypeStruct((b, d), jnp.float32),
        in_specs=[pl.BlockSpec(memory_space=pltpu.VMEM)] * 7,
        out_specs=pl.BlockSpec(memory_space=pltpu.VMEM),
        scratch_shapes=[
            pltpu.VMEM((N_EXCH, b, d), jnp.bfloat16),
            pltpu.VMEM((N_EXCH, b, d), jnp.bfloat16),
            pltpu.SemaphoreType.DMA((N_EXCH,)),
            pltpu.SemaphoreType.DMA((N_EXCH,)),
        ],
    )(x, Win0, Wout0, Win1, Wout1, Win2, Wout2)


# device time: 35402 ns/iter; 1.7069x vs baseline; 1.2257x over previous
import jax
import jax.numpy as jnp
from jax import lax
from jax.experimental import pallas as pl
from jax.experimental.pallas import tpu as pltpu

N_DEV = 4
N_LAYERS = 3
N_CHUNK = 2
N_SLOTS = N_LAYERS * 2 * N_CHUNK


def kernel(x, Win0, Wout0, Win1, Wout1, Win2, Wout2):
    b, d = x.shape
    rows = b // N_CHUNK

    def body(x_ref, win0, wout0, win1, wout1, win2, wout2,
             out_ref, send_ref, comm_ref, send_sems, recv_sems):
        my_pos = lax.axis_index("i")
        partners = [my_pos ^ 1, 3 - my_pos]

        barrier = pltpu.get_barrier_semaphore()
        for p_ in partners:
            pltpu.semaphore_signal(
                barrier, 1, device_id=(p_,),
                device_id_type=pl.DeviceIdType.MESH,
            )
        pltpu.semaphore_wait(barrier, 2)

        wins = [win0, win1, win2]
        wouts = [wout0, wout1, wout2]

        rdmas = {}

        def exch_start(l, s, c, data_f32):
            e = 4 * l + 2 * s + c
            send_ref[e] = data_f32.astype(jnp.bfloat16)
            r = pltpu.make_async_remote_copy(
                src_ref=send_ref.at[e],
                dst_ref=comm_ref.at[e],
                send_sem=send_sems.at[e],
                recv_sem=recv_sems.at[e],
                device_id=(partners[s],),
                device_id_type=pl.DeviceIdType.MESH,
            )
            r.start()
            rdmas[e] = r

        def exch_recv(l, s, c):
            e = 4 * l + 2 * s + c
            rdmas[e].wait_recv()
            return comm_ref[e].astype(jnp.float32)

        x_ch = [
            x_ref[pl.ds(c * rows, rows), :].astype(jnp.bfloat16)
            for c in range(N_CHUNK)
        ]
        p = [None] * N_CHUNK
        s1 = [None] * N_CHUNK

        for l in range(N_LAYERS):
            w_in = wins[l][...].astype(jnp.bfloat16)
            w_out = wouts[l][...].astype(jnp.bfloat16)
            for c in range(N_CHUNK):
                h = jnp.maximum(
                    jnp.dot(x_ch[c], w_in, preferred_element_type=jnp.float32),
                    0.0,
                ).astype(jnp.bfloat16)
                p[c] = jnp.dot(h, w_out, preferred_element_type=jnp.float32)
                exch_start(l, 0, c, p[c])
            for c in range(N_CHUNK):
                s1[c] = p[c] + exch_recv(l, 0, c)
                exch_start(l, 1, c, s1[c])
            for c in range(N_CHUNK):
                tot = s1[c] + exch_recv(l, 1, c)
                if l == N_LAYERS - 1:
                    out_ref[pl.ds(c * rows, rows), :] = tot
                else:
                    x_ch[c] = tot.astype(jnp.bfloat16)

        for r in rdmas.values():
            r.wait_send()

    return pl.pallas_call(
        body,
        out_shape=jax.ShapeDtypeStruct((b, d), jnp.float32),
        in_specs=[pl.BlockSpec(memory_space=pltpu.VMEM)] * 7,
        out_specs=pl.BlockSpec(memory_space=pltpu.VMEM),
        scratch_shapes=[
            pltpu.VMEM((N_SLOTS, rows, d), jnp.bfloat16),
            pltpu.VMEM((N_SLOTS, rows, d), jnp.bfloat16),
            pltpu.SemaphoreType.DMA((N_SLOTS,)),
            pltpu.SemaphoreType.DMA((N_SLOTS,)),
        ],
        compiler_params=pltpu.CompilerParams(collective_id=0),
    )(x, Win0, Wout0, Win1, Wout1, Win2, Wout2)


# device time: 32621 ns/iter; 1.8524x vs baseline; 1.0853x over previous
import jax
import jax.numpy as jnp
from jax import lax
from jax.experimental import pallas as pl
from jax.experimental.pallas import tpu as pltpu

N_DEV = 4
N_LAYERS = 3
N_CHUNK = 2
N_SLOTS = N_LAYERS * 2 * N_CHUNK


def kernel(x, Win0, Wout0, Win1, Wout1, Win2, Wout2):
    b, d = x.shape
    rows = b // N_CHUNK

    def body(x_ref, win0, wout0, win1, wout1, win2, wout2,
             out_ref, send_ref, comm_ref, send_sems, recv_sems):
        my_pos = lax.axis_index("i")
        partners = [my_pos ^ 1, 3 - my_pos]

        barrier = pltpu.get_barrier_semaphore()
        for p_ in partners:
            pltpu.semaphore_signal(
                barrier, 1, device_id=(p_,),
                device_id_type=pl.DeviceIdType.MESH,
            )
        pltpu.semaphore_wait(barrier, 2)

        wins = [win0, win1, win2]
        wouts = [wout0, wout1, wout2]

        rdmas = {}

        def exch_start(l, s, c, data_f32):
            e = 4 * l + 2 * s + c
            send_ref[e] = data_f32.astype(jnp.bfloat16)
            r = pltpu.make_async_remote_copy(
                src_ref=send_ref.at[e],
                dst_ref=comm_ref.at[e],
                send_sem=send_sems.at[e],
                recv_sem=recv_sems.at[e],
                device_id=(partners[s],),
                device_id_type=pl.DeviceIdType.MESH,
            )
            r.start()
            rdmas[e] = r

        def exch_recv(l, s, c):
            e = 4 * l + 2 * s + c
            rdmas[e].wait_recv()
            return comm_ref[e].astype(jnp.float32)

        w_cache = {}

        def layer_compute(l, x_bf):
            if l not in w_cache:
                w_cache[l] = (
                    wins[l][...].astype(jnp.bfloat16),
                    wouts[l][...].astype(jnp.bfloat16),
                )
            w_in, w_out = w_cache[l]
            h = jnp.maximum(
                jnp.dot(x_bf, w_in, preferred_element_type=jnp.float32), 0.0
            ).astype(jnp.bfloat16)
            return jnp.dot(h, w_out, preferred_element_type=jnp.float32)

        p = [None] * N_CHUNK
        s1 = [None] * N_CHUNK

        for c in range(N_CHUNK):
            x_bf = x_ref[pl.ds(c * rows, rows), :].astype(jnp.bfloat16)
            p[c] = layer_compute(0, x_bf)
            exch_start(0, 0, c, p[c])

        for l in range(N_LAYERS):
            for c in range(N_CHUNK):
                s1[c] = p[c] + exch_recv(l, 0, c)
                exch_start(l, 1, c, s1[c])
            for c in range(N_CHUNK):
                tot = s1[c] + exch_recv(l, 1, c)
                if l == N_LAYERS - 1:
                    out_ref[pl.ds(c * rows, rows), :] = tot
                else:
                    p[c] = layer_compute(l + 1, tot.astype(jnp.bfloat16))
                    exch_start(l + 1, 0, c, p[c])

        for r in rdmas.values():
            r.wait_send()

    return pl.pallas_call(
        body,
        out_shape=jax.ShapeDtypeStruct((b, d), jnp.float32),
        in_specs=[pl.BlockSpec(memory_space=pltpu.VMEM)] * 7,
        out_specs=pl.BlockSpec(memory_space=pltpu.VMEM),
        scratch_shapes=[
            pltpu.VMEM((N_SLOTS, rows, d), jnp.bfloat16),
            pltpu.VMEM((N_SLOTS, rows, d), jnp.bfloat16),
            pltpu.SemaphoreType.DMA((N_SLOTS,)),
            pltpu.SemaphoreType.DMA((N_SLOTS,)),
        ],
        compiler_params=pltpu.CompilerParams(collective_id=0),
    )(x, Win0, Wout0, Win1, Wout1, Win2, Wout2)


# device time: 32613 ns/iter; 1.8528x vs baseline; 1.0002x over previous
import jax
import jax.numpy as jnp
from jax import lax
from jax.experimental import pallas as pl
from jax.experimental.pallas import tpu as pltpu

N_DEV = 4
N_LAYERS = 3
N_CHUNK = 2
N_SLOTS = N_LAYERS * 2 * N_CHUNK


def kernel(x, Win0, Wout0, Win1, Wout1, Win2, Wout2):
    b, d = x.shape
    rows = b // N_CHUNK

    def body(x_ref, win0, wout0, win1, wout1, win2, wout2,
             out_ref, send_ref, comm_ref, send_sems, recv_sems):
        my_pos = lax.axis_index("i")
        partners = [my_pos ^ 1, 3 - my_pos]

        barrier = pltpu.get_barrier_semaphore()
        for p_ in partners:
            pltpu.semaphore_signal(
                barrier, 1, device_id=(p_,),
                device_id_type=pl.DeviceIdType.MESH,
            )
        pltpu.semaphore_wait(barrier, 2)

        wins = [win0, win1, win2]
        wouts = [wout0, wout1, wout2]

        rdmas = {}

        def exch_start(l, s, c, data_f32):
            e = 4 * l + 2 * s + c
            send_ref[e] = data_f32.astype(jnp.bfloat16)
            r = pltpu.make_async_remote_copy(
                src_ref=send_ref.at[e],
                dst_ref=comm_ref.at[e],
                send_sem=send_sems.at[e],
                recv_sem=recv_sems.at[e],
                device_id=(partners[s],),
                device_id_type=pl.DeviceIdType.MESH,
            )
            r.start()
            rdmas[e] = r

        def exch_recv(l, s, c):
            e = 4 * l + 2 * s + c
            rdmas[e].wait_recv()
            return comm_ref[e].astype(jnp.float32)

        w_cache = {}

        def get_w(l):
            if l not in w_cache:
                w_cache[l] = (
                    wins[l][...].astype(jnp.bfloat16),
                    wouts[l][...].astype(jnp.bfloat16),
                )
            return w_cache[l]

        def layer_compute(l, x_bf):
            w_in, w_out = get_w(l)
            h = jnp.maximum(
                jnp.dot(x_bf, w_in, preferred_element_type=jnp.float32), 0.0
            ).astype(jnp.bfloat16)
            return jnp.dot(h, w_out, preferred_element_type=jnp.float32)

        p = [None] * N_CHUNK
        s1 = [None] * N_CHUNK

        for c in range(N_CHUNK):
            x_bf = x_ref[pl.ds(c * rows, rows), :].astype(jnp.bfloat16)
            p[c] = layer_compute(0, x_bf)
            exch_start(0, 0, c, p[c])
        if N_LAYERS > 1:
            get_w(1)

        for l in range(N_LAYERS):
            for c in range(N_CHUNK):
                s1[c] = p[c] + exch_recv(l, 0, c)
                exch_start(l, 1, c, s1[c])
            if l + 2 <= N_LAYERS - 1:
                get_w(l + 2)
            for c in range(N_CHUNK):
                tot = s1[c] + exch_recv(l, 1, c)
                if l == N_LAYERS - 1:
                    out_ref[pl.ds(c * rows, rows), :] = tot
                else:
                    p[c] = layer_compute(l + 1, tot.astype(jnp.bfloat16))
                    exch_start(l + 1, 0, c, p[c])

        for r in rdmas.values():
            r.wait_send()

    return pl.pallas_call(
        body,
        out_shape=jax.ShapeDtypeStruct((b, d), jnp.float32),
        in_specs=[pl.BlockSpec(memory_space=pltpu.VMEM)] * 7,
        out_specs=pl.BlockSpec(memory_space=pltpu.VMEM),
        scratch_shapes=[
            pltpu.VMEM((N_SLOTS, rows, d), jnp.bfloat16),
            pltpu.VMEM((N_SLOTS, rows, d), jnp.bfloat16),
            pltpu.SemaphoreType.DMA((N_SLOTS,)),
            pltpu.SemaphoreType.DMA((N_SLOTS,)),
        ],
        compiler_params=pltpu.CompilerParams(collective_id=0),
    )(x, Win0, Wout0, Win1, Wout1, Win2, Wout2)


# device time: 28647 ns/iter; 2.1093x vs baseline; 1.1384x over previous
import jax
import jax.numpy as jnp
from jax import lax
from jax.experimental import pallas as pl
from jax.experimental.pallas import tpu as pltpu

N_DEV = 4
N_LAYERS = 3
N_CHUNK = 4
N_SLOTS = N_LAYERS * 2 * N_CHUNK


def kernel(x, Win0, Wout0, Win1, Wout1, Win2, Wout2):
    b, d = x.shape
    rows = b // N_CHUNK

    def body(x_ref, win0, wout0, win1, wout1, win2, wout2,
             out_ref, send_ref, comm_ref, send_sems, recv_sems):
        my_pos = lax.axis_index("i")
        partners = [my_pos ^ 1, 3 - my_pos]

        barrier = pltpu.get_barrier_semaphore()
        for p_ in partners:
            pltpu.semaphore_signal(
                barrier, 1, device_id=(p_,),
                device_id_type=pl.DeviceIdType.MESH,
            )
        pltpu.semaphore_wait(barrier, 2)

        wins = [win0, win1, win2]
        wouts = [wout0, wout1, wout2]

        rdmas = {}

        def exch_start(l, s, c, data_f32):
            e = (l * 2 + s) * N_CHUNK + c
            send_ref[e] = data_f32.astype(jnp.bfloat16)
            r = pltpu.make_async_remote_copy(
                src_ref=send_ref.at[e],
                dst_ref=comm_ref.at[e],
                send_sem=send_sems.at[e],
                recv_sem=recv_sems.at[e],
                device_id=(partners[s],),
                device_id_type=pl.DeviceIdType.MESH,
            )
            r.start()
            rdmas[e] = r

        def exch_recv(l, s, c):
            e = (l * 2 + s) * N_CHUNK + c
            rdmas[e].wait_recv()
            return comm_ref[e].astype(jnp.float32)

        w_cache = {}

        def get_w(l):
            if l not in w_cache:
                w_cache[l] = (
                    wins[l][...].astype(jnp.bfloat16),
                    wouts[l][...].astype(jnp.bfloat16),
                )
            return w_cache[l]

        def layer_compute(l, x_bf):
            w_in, w_out = get_w(l)
            h = jnp.maximum(
                jnp.dot(x_bf, w_in, preferred_element_type=jnp.float32), 0.0
            ).astype(jnp.bfloat16)
            return jnp.dot(h, w_out, preferred_element_type=jnp.float32)

        p = [None] * N_CHUNK
        s1 = [None] * N_CHUNK

        for c in range(N_CHUNK):
            x_bf = x_ref[pl.ds(c * rows, rows), :].astype(jnp.bfloat16)
            p[c] = layer_compute(0, x_bf)
            exch_start(0, 0, c, p[c])
        if N_LAYERS > 1:
            get_w(1)

        for l in range(N_LAYERS):
            for c in range(N_CHUNK):
                s1[c] = p[c] + exch_recv(l, 0, c)
                exch_start(l, 1, c, s1[c])
            if l + 2 <= N_LAYERS - 1:
                get_w(l + 2)
            for c in range(N_CHUNK):
                tot = s1[c] + exch_recv(l, 1, c)
                if l == N_LAYERS - 1:
                    out_ref[pl.ds(c * rows, rows), :] = tot
                else:
                    p[c] = layer_compute(l + 1, tot.astype(jnp.bfloat16))
                    exch_start(l + 1, 0, c, p[c])

        for r in rdmas.values():
            r.wait_send()

    return pl.pallas_call(
        body,
        out_shape=jax.ShapeDtypeStruct((b, d), jnp.float32),
        in_specs=[pl.BlockSpec(memory_space=pltpu.VMEM)] * 7,
        out_specs=pl.BlockSpec(memory_space=pltpu.VMEM),
        scratch_shapes=[
            pltpu.VMEM((N_SLOTS, rows, d), jnp.bfloat16),
            pltpu.VMEM((N_SLOTS, rows, d), jnp.bfloat16),
            pltpu.SemaphoreType.DMA((N_SLOTS,)),
            pltpu.SemaphoreType.DMA((N_SLOTS,)),
        ],
        compiler_params=pltpu.CompilerParams(collective_id=0),
    )(x, Win0, Wout0, Win1, Wout1, Win2, Wout2)
